# baseline (speedup 1.0000x reference)
import functools

import jax
import jax.numpy as jnp
from jax import lax
from jax.experimental import pallas as pl
from jax.experimental.pallas import tpu as pltpu
from jax.experimental.pallas import tpu_sc as plsc

B, S, HID = 1, 2048, 2048
H, D, ROPE, NOPE, QLORA, TOPK = 16, 128, 64, 64, 1536, 2048


def _hadamard_transform(x, scale):
    shp = x.shape
    n = shp[-1]
    y = x.reshape(-1, n)
    h = 1
    while h < n:
        y = y.reshape(-1, n // (2 * h), 2, h)
        a = y[:, :, 0, :]
        b = y[:, :, 1, :]
        y = jnp.stack([a + b, a - b], axis=2)
        y = y.reshape(-1, n)
        h *= 2
    return (y * scale).reshape(shp)


def _rotate_activation(x):
    xb = x.astype(jnp.bfloat16)
    return _hadamard_transform(xb, xb.shape[-1] ** (-0.5))


def _apply_rope(x, angles):
    cos = jnp.cos(angles)
    sin = jnp.sin(angles)
    if x.ndim == 4:
        cos = cos[None, :, None, :]
        sin = sin[None, :, None, :]
    else:
        cos = cos[None, :, :]
        sin = sin[None, :, :]
    xr = x[..., 0::2].astype(jnp.float32)
    xi = x[..., 1::2].astype(jnp.float32)
    yr = xr * cos - xi * sin
    yi = xr * sin + xi * cos
    y = jnp.stack([yr, yi], axis=-1).reshape(x.shape)
    return y.astype(x.dtype)


def _layer_norm(x, g, b, eps=1e-5):
    m = jnp.mean(x, axis=-1, keepdims=True)
    v = jnp.var(x, axis=-1, keepdims=True)
    return (x - m) / jnp.sqrt(v + eps) * g + b


BQ = 512


def _scores_kernel(qf_ref, kft_ref, w_ref, out_ref):
    h = pl.program_id(1)
    s = lax.dot_general(qf_ref[0], kft_ref[...],
                        (((1,), (0,)), ((), ())),
                        preferred_element_type=jnp.float32)
    s = jnp.maximum(s, 0.0) * w_ref[0, 0][:, None]

    @pl.when(h == 0)
    def _():
        out_ref[...] = s

    @pl.when(h > 0)
    def _():
        out_ref[...] += s


def _scores(qf_h, kf_t, w_h):
    return pl.pallas_call(
        _scores_kernel,
        grid=(S // BQ, H),
        in_specs=[
            pl.BlockSpec((1, BQ, D), lambda i, h: (h, i, 0)),
            pl.BlockSpec((D, S), lambda i, h: (0, 0)),
            pl.BlockSpec((1, 1, BQ), lambda i, h: (h, 0, i)),
        ],
        out_specs=pl.BlockSpec((BQ, S), lambda i, h: (i, 0)),
        out_shape=jax.ShapeDtypeStruct((S, S), jnp.float32),
    )(qf_h, kf_t, w_h)


NW = 32
RPT = S // NW
NGRP = RPT // 16


def _argsort_body(scores_hbm, out_hbm, rowbuf, keys, ia, hist):
    lane = lax.iota(jnp.int32, 16)
    zero16 = jnp.zeros((16,), jnp.int32)
    one16 = jnp.ones((16,), jnp.int32)

    def full(v):
        return jnp.full((16,), v, jnp.int32)

    wid = lax.axis_index("s") * 2 + lax.axis_index("c")

    def group(g, _):
        q0 = wid * RPT + g * 16

        def dmain(r, _):
            pltpu.sync_copy(scores_hbm.at[q0 + r], rowbuf.at[pl.ds(r * S, S)])
            return 0

        lax.fori_loop(0, 16, dmain, 0)

        def tbody(i, _):
            u = plsc.load_gather(rowbuf, [lane * S + full(i)])
            m = lax.shift_right_arithmetic(u, 31)
            xorv = jnp.bitwise_xor(
                jnp.bitwise_or(m, jnp.int32(-2147483648)), jnp.int32(-1))
            keys[pl.ds(i * 16, 16)] = jnp.bitwise_xor(u, xorv)
            return 0

        lax.fori_loop(0, S, tbody, 0)

        for p, (src, dst) in enumerate(
                [(None, ia), (ia, rowbuf), (rowbuf, ia), (ia, None)]):
            sh = 8 * p

            def zbody(b, _):
                hist[pl.ds(b * 16, 16)] = zero16
                return 0

            lax.fori_loop(0, 256, zbody, 0)

            def hbody(i, _, src=src, sh=sh):
                if src is None:
                    kv = keys[pl.ds(i * 16, 16)]
                else:
                    ix = src[pl.ds(i * 16, 16)]
                    kv = plsc.load_gather(keys, [ix * 16 + lane])
                d = jnp.bitwise_and(lax.shift_right_logical(kv, sh), 255)
                plsc.addupdate_scatter(hist, [d * 16 + lane], one16)
                return 0

            lax.fori_loop(0, S, hbody, 0)

            def sbody(b, run):
                h = hist[pl.ds(b * 16, 16)]
                hist[pl.ds(b * 16, 16)] = run
                return run + h

            lax.fori_loop(0, 256, sbody, zero16)

            def pbody(i, _, src=src, dst=dst, sh=sh):
                if src is None:
                    ix = full(i)
                    kv = keys[pl.ds(i * 16, 16)]
                else:
                    ix = src[pl.ds(i * 16, 16)]
                    kv = plsc.load_gather(keys, [ix * 16 + lane])
                d = jnp.bitwise_and(lax.shift_right_logical(kv, sh), 255)
                off = plsc.load_gather(hist, [d * 16 + lane])
                if dst is None:
                    plsc.store_scatter(rowbuf, [lane * S + off], ix)
                else:
                    plsc.store_scatter(dst, [off * 16 + lane], ix)
                plsc.addupdate_scatter(hist, [d * 16 + lane], one16)
                return 0

            lax.fori_loop(0, S, pbody, 0)

        def dmaout(r, _):
            pltpu.sync_copy(rowbuf.at[pl.ds(r * S, S)], out_hbm.at[q0 + r])
            return 0

        lax.fori_loop(0, 16, dmaout, 0)
        return 0

    lax.fori_loop(0, NGRP, group, 0)


def _argsort_desc(scores_bits):
    f = pl.kernel(
        _argsort_body,
        out_type=jax.ShapeDtypeStruct((S, S), jnp.int32),
        mesh=plsc.VectorSubcoreMesh(core_axis_name="c", subcore_axis_name="s"),
        scratch_types=[
            pltpu.VMEM((16 * S,), jnp.int32),
            pltpu.VMEM((16 * S,), jnp.int32),
            pltpu.VMEM((16 * S,), jnp.int32),
            pltpu.VMEM((256 * 16,), jnp.int32),
        ],
        compiler_params=pltpu.CompilerParams(needs_layout_passes=False),
    )
    return f(scores_bits)


def kernel(x, q_resid, freqs_cis, Wq_b, Wk, k_norm_weight, k_norm_bias, Wweights):
    softmax_scale = D ** (-0.5)
    q = (q_resid @ Wq_b.T).reshape(B, S, H, D)
    q_nope, q_pe = q[..., :NOPE], q[..., NOPE:]
    k = _layer_norm(x @ Wk.T, k_norm_weight, k_norm_bias)
    k_nope, k_pe = k[..., :NOPE], k[..., NOPE:]
    q_pe = _apply_rope(q_pe, freqs_cis)
    k_pe = _apply_rope(k_pe, freqs_cis)
    q = jnp.concatenate([q_nope, q_pe], axis=-1)
    k = jnp.concatenate([k_nope, k_pe], axis=-1)
    q = _rotate_activation(q)
    k = _rotate_activation(k)
    weights = (x @ Wweights.T).astype(jnp.float32) * (H ** (-0.5)) * softmax_scale

    qf_h = jnp.transpose(q[0], (1, 0, 2))
    kf_t = jnp.transpose(k[0], (1, 0))
    w_h = jnp.transpose(weights[0], (1, 0))[:, None, :]

    sc = _scores(qf_h, kf_t, w_h)
    scores_bits = lax.bitcast_convert_type(sc, jnp.int32)
    topk_indices = _argsort_desc(scores_bits)
    return topk_indices[None]


# speedup vs baseline: 1.0957x; 1.0957x over previous
import functools

import jax
import jax.numpy as jnp
from jax import lax
from jax.experimental import pallas as pl
from jax.experimental.pallas import tpu as pltpu
from jax.experimental.pallas import tpu_sc as plsc

B, S, HID = 1, 2048, 2048
H, D, ROPE, NOPE, QLORA, TOPK = 16, 128, 64, 64, 1536, 2048


def _hadamard_transform(x, scale):
    shp = x.shape
    n = shp[-1]
    y = x.reshape(-1, n)
    h = 1
    while h < n:
        y = y.reshape(-1, n // (2 * h), 2, h)
        a = y[:, :, 0, :]
        b = y[:, :, 1, :]
        y = jnp.stack([a + b, a - b], axis=2)
        y = y.reshape(-1, n)
        h *= 2
    return (y * scale).reshape(shp)


def _rotate_activation(x):
    xb = x.astype(jnp.bfloat16)
    return _hadamard_transform(xb, xb.shape[-1] ** (-0.5))


def _apply_rope(x, angles):
    cos = jnp.cos(angles)
    sin = jnp.sin(angles)
    if x.ndim == 4:
        cos = cos[None, :, None, :]
        sin = sin[None, :, None, :]
    else:
        cos = cos[None, :, :]
        sin = sin[None, :, :]
    xr = x[..., 0::2].astype(jnp.float32)
    xi = x[..., 1::2].astype(jnp.float32)
    yr = xr * cos - xi * sin
    yi = xr * sin + xi * cos
    y = jnp.stack([yr, yi], axis=-1).reshape(x.shape)
    return y.astype(x.dtype)


def _layer_norm(x, g, b, eps=1e-5):
    m = jnp.mean(x, axis=-1, keepdims=True)
    v = jnp.var(x, axis=-1, keepdims=True)
    return (x - m) / jnp.sqrt(v + eps) * g + b


BQ = 512


def _scores_kernel(qf_ref, kft_ref, w_ref, out_ref):
    h = pl.program_id(1)
    s = lax.dot_general(qf_ref[0], kft_ref[...],
                        (((1,), (0,)), ((), ())),
                        preferred_element_type=jnp.float32)
    s = jnp.maximum(s, 0.0) * w_ref[0, 0][:, None]

    @pl.when(h == 0)
    def _():
        out_ref[...] = s

    @pl.when(h > 0)
    def _():
        out_ref[...] += s


def _scores(qf_h, kf_t, w_h):
    return pl.pallas_call(
        _scores_kernel,
        grid=(S // BQ, H),
        in_specs=[
            pl.BlockSpec((1, BQ, D), lambda i, h: (h, i, 0)),
            pl.BlockSpec((D, S), lambda i, h: (0, 0)),
            pl.BlockSpec((1, 1, BQ), lambda i, h: (h, 0, i)),
        ],
        out_specs=pl.BlockSpec((BQ, S), lambda i, h: (i, 0)),
        out_shape=jax.ShapeDtypeStruct((S, S), jnp.float32),
    )(qf_h, kf_t, w_h)


NW = 32
RPT = S // NW
NGRP = RPT // 16


NCH = 4
CH = S // NCH
RB = S


def _argsort_body(scores_hbm, out_hbm, rowbuf, keys, ia, h0, h1, h2, h3, sem):
    lane = lax.iota(jnp.int32, 16)
    zero16 = jnp.zeros((16,), jnp.int32)
    one16 = jnp.ones((16,), jnp.int32)
    hists = [h0, h1, h2, h3]

    def full(v):
        return jnp.full((16,), v, jnp.int32)

    wid = lax.axis_index("s") * 2 + lax.axis_index("c")

    def group(g, _):
        q0 = wid * RPT + g * 16

        copies = [
            pltpu.async_copy(scores_hbm.at[q0 + r],
                             rowbuf.at[pl.ds(r * RB, S)], sem)
            for r in range(16)
        ]
        for c in copies:
            c.wait()

        def tbody(i, _):
            for c in range(NCH):
                jv = jnp.bitwise_and(full(i + c * CH) + lane, S - 1)
                u = plsc.load_gather(rowbuf, [lane * RB + jv])
                m = lax.shift_right_arithmetic(u, 31)
                xorv = jnp.bitwise_xor(
                    jnp.bitwise_or(m, jnp.int32(-2147483648)), jnp.int32(-1))
                plsc.store_scatter(keys, [jv * 16 + lane],
                                   jnp.bitwise_xor(u, xorv))
            return 0

        lax.fori_loop(0, CH, tbody, 0)

        for p, (src, dst) in enumerate(
                [(None, ia), (ia, rowbuf), (rowbuf, ia), (ia, None)]):
            sh = 8 * p

            def digit(kv, sh=sh):
                d = lax.shift_right_logical(kv, sh) if sh else kv
                return jnp.bitwise_and(d, 255) if sh < 24 else d

            def zbody(b, _):
                for hc in hists:
                    hc[pl.ds(b * 16, 16)] = zero16
                return 0

            lax.fori_loop(0, 256, zbody, 0)

            def hbody(i, _, src=src):
                for c in range(NCH):
                    j = i + c * CH
                    if src is None:
                        kv = keys[pl.ds(j * 16, 16)]
                    else:
                        ix = src[pl.ds(j * 16, 16)]
                        kv = plsc.load_gather(keys, [ix * 16 + lane])
                    plsc.addupdate_scatter(hists[c], [digit(kv) * 16 + lane],
                                           one16)
                return 0

            lax.fori_loop(0, CH, hbody, 0)

            def sbody(b, run):
                hs = [hc[pl.ds(b * 16, 16)] for hc in hists]
                for c, hc in enumerate(hists):
                    hc[pl.ds(b * 16, 16)] = run
                    run = run + hs[c]
                return run

            lax.fori_loop(0, 256, sbody, zero16)

            def pbody(i, _, src=src, dst=dst):
                for c in range(NCH):
                    j = i + c * CH
                    if src is None:
                        ix = full(j)
                        kv = keys[pl.ds(j * 16, 16)]
                    else:
                        ix = src[pl.ds(j * 16, 16)]
                        kv = plsc.load_gather(keys, [ix * 16 + lane])
                    dd = digit(kv) * 16 + lane
                    off = plsc.load_gather(hists[c], [dd])
                    if dst is None:
                        plsc.store_scatter(rowbuf, [lane * RB + off], ix)
                    else:
                        plsc.store_scatter(dst, [off * 16 + lane], ix)
                    plsc.addupdate_scatter(hists[c], [dd], one16)
                return 0

            lax.fori_loop(0, CH, pbody, 0)

        copies = [
            pltpu.async_copy(rowbuf.at[pl.ds(r * RB, S)],
                             out_hbm.at[q0 + r], sem)
            for r in range(16)
        ]
        for c in copies:
            c.wait()
        return 0

    lax.fori_loop(0, NGRP, group, 0)


def _argsort_desc(scores_bits):
    f = pl.kernel(
        _argsort_body,
        out_type=jax.ShapeDtypeStruct((S, S), jnp.int32),
        mesh=plsc.VectorSubcoreMesh(core_axis_name="c", subcore_axis_name="s"),
        scratch_types=[
            pltpu.VMEM((16 * S,), jnp.int32),
            pltpu.VMEM((16 * S,), jnp.int32),
            pltpu.VMEM((16 * S,), jnp.int32),
            pltpu.VMEM((256 * 16,), jnp.int32),
            pltpu.VMEM((256 * 16,), jnp.int32),
            pltpu.VMEM((256 * 16,), jnp.int32),
            pltpu.VMEM((256 * 16,), jnp.int32),
            pltpu.SemaphoreType.DMA,
        ],
        compiler_params=pltpu.CompilerParams(needs_layout_passes=False),
    )
    return f(scores_bits)


def kernel(x, q_resid, freqs_cis, Wq_b, Wk, k_norm_weight, k_norm_bias, Wweights):
    softmax_scale = D ** (-0.5)
    q = (q_resid @ Wq_b.T).reshape(B, S, H, D)
    q_nope, q_pe = q[..., :NOPE], q[..., NOPE:]
    k = _layer_norm(x @ Wk.T, k_norm_weight, k_norm_bias)
    k_nope, k_pe = k[..., :NOPE], k[..., NOPE:]
    q_pe = _apply_rope(q_pe, freqs_cis)
    k_pe = _apply_rope(k_pe, freqs_cis)
    q = jnp.concatenate([q_nope, q_pe], axis=-1)
    k = jnp.concatenate([k_nope, k_pe], axis=-1)
    q = _rotate_activation(q)
    k = _rotate_activation(k)
    weights = (x @ Wweights.T).astype(jnp.float32) * (H ** (-0.5)) * softmax_scale

    qf_h = jnp.transpose(q[0], (1, 0, 2))
    kf_t = jnp.transpose(k[0], (1, 0))
    w_h = jnp.transpose(weights[0], (1, 0))[:, None, :]

    sc = _scores(qf_h, kf_t, w_h)
    scores_bits = lax.bitcast_convert_type(sc, jnp.int32)
    topk_indices = _argsort_desc(scores_bits)
    return topk_indices[None]


# speedup vs baseline: 1.9289x; 1.7604x over previous
import functools

import jax
import jax.numpy as jnp
from jax import lax
from jax.experimental import pallas as pl
from jax.experimental.pallas import tpu as pltpu
from jax.experimental.pallas import tpu_sc as plsc

B, S, HID = 1, 2048, 2048
H, D, ROPE, NOPE, QLORA, TOPK = 16, 128, 64, 64, 1536, 2048


def _hadamard_transform(x, scale):
    shp = x.shape
    n = shp[-1]
    y = x.reshape(-1, n)
    h = 1
    while h < n:
        y = y.reshape(-1, n // (2 * h), 2, h)
        a = y[:, :, 0, :]
        b = y[:, :, 1, :]
        y = jnp.stack([a + b, a - b], axis=2)
        y = y.reshape(-1, n)
        h *= 2
    return (y * scale).reshape(shp)


def _rotate_activation(x):
    xb = x.astype(jnp.bfloat16)
    return _hadamard_transform(xb, xb.shape[-1] ** (-0.5))


def _apply_rope(x, angles):
    cos = jnp.cos(angles)
    sin = jnp.sin(angles)
    if x.ndim == 4:
        cos = cos[None, :, None, :]
        sin = sin[None, :, None, :]
    else:
        cos = cos[None, :, :]
        sin = sin[None, :, :]
    xr = x[..., 0::2].astype(jnp.float32)
    xi = x[..., 1::2].astype(jnp.float32)
    yr = xr * cos - xi * sin
    yi = xr * sin + xi * cos
    y = jnp.stack([yr, yi], axis=-1).reshape(x.shape)
    return y.astype(x.dtype)


def _layer_norm(x, g, b, eps=1e-5):
    m = jnp.mean(x, axis=-1, keepdims=True)
    v = jnp.var(x, axis=-1, keepdims=True)
    return (x - m) / jnp.sqrt(v + eps) * g + b


BQ = 512


def _scores_kernel(qf_ref, kft_ref, w_ref, out_ref):
    h = pl.program_id(1)
    s = lax.dot_general(qf_ref[0], kft_ref[...],
                        (((1,), (0,)), ((), ())),
                        preferred_element_type=jnp.float32)
    s = jnp.maximum(s, 0.0) * w_ref[0, 0][:, None]

    @pl.when(h == 0)
    def _():
        out_ref[...] = s

    @pl.when(h > 0)
    def _():
        out_ref[...] += s


def _scores(qf_h, kf_t, w_h):
    return pl.pallas_call(
        _scores_kernel,
        grid=(S // BQ, H),
        in_specs=[
            pl.BlockSpec((1, BQ, D), lambda i, h: (h, i, 0)),
            pl.BlockSpec((D, S), lambda i, h: (0, 0)),
            pl.BlockSpec((1, 1, BQ), lambda i, h: (h, 0, i)),
        ],
        out_specs=pl.BlockSpec((BQ, S), lambda i, h: (i, 0)),
        out_shape=jax.ShapeDtypeStruct((S, S), jnp.float32),
    )(qf_h, kf_t, w_h)


NW = 32
RPT = S // NW
NGRP = RPT // 16


NCH = 4
CH = S // NCH
RB = S


def _argsort_body(scores_hbm, out_hbm, rowbuf, keys, ia, h0, h1, h2, h3, sem):
    lane = lax.iota(jnp.int32, 16)
    zero16 = jnp.zeros((16,), jnp.int32)
    one16 = jnp.ones((16,), jnp.int32)
    hists = [h0, h1, h2, h3]

    def full(v):
        return jnp.full((16,), v, jnp.int32)

    wid = lax.axis_index("s") * 2 + lax.axis_index("c")

    def group(g, _):
        q0 = wid * RPT + g * 16

        copies = [
            pltpu.async_copy(scores_hbm.at[q0 + r],
                             rowbuf.at[pl.ds(r * RB, S)], sem)
            for r in range(16)
        ]
        for c in copies:
            c.wait()

        def tbody(i, _):
            jvs = [jnp.bitwise_and(full(i + c * CH) + lane, S - 1)
                   for c in range(NCH)]
            us = [plsc.load_gather(rowbuf, [lane * RB + jv]) for jv in jvs]
            ks = []
            for u in us:
                m = lax.shift_right_arithmetic(u, 31)
                xorv = jnp.bitwise_xor(
                    jnp.bitwise_or(m, jnp.int32(-2147483648)), jnp.int32(-1))
                ks.append(jnp.bitwise_xor(u, xorv))
            for jv, kv in zip(jvs, ks):
                plsc.store_scatter(keys, [jv * 16 + lane], kv)
            return 0

        lax.fori_loop(0, CH, tbody, 0)

        for p, (src, dst) in enumerate(
                [(None, ia), (ia, rowbuf), (rowbuf, ia), (ia, None)]):
            sh = 8 * p

            def digit(kv, sh=sh):
                d = lax.shift_right_logical(kv, sh) if sh else kv
                return jnp.bitwise_and(d, 255) if sh < 24 else d

            def zbody(b, _):
                for hc in hists:
                    hc[pl.ds(b * 16, 16)] = zero16
                return 0

            lax.fori_loop(0, 256, zbody, 0)

            def hbody(i, _, src=src):
                if src is None:
                    kvs = [keys[pl.ds((i + c * CH) * 16, 16)]
                           for c in range(NCH)]
                else:
                    ixs = [src[pl.ds((i + c * CH) * 16, 16)]
                           for c in range(NCH)]
                    kvs = [plsc.load_gather(keys, [ix * 16 + lane])
                           for ix in ixs]
                dds = [digit(kv) * 16 + lane for kv in kvs]
                for c in range(NCH):
                    plsc.addupdate_scatter(hists[c], [dds[c]], one16)
                return 0

            lax.fori_loop(0, CH, hbody, 0)

            def sbody(b, run):
                hs = [hc[pl.ds(b * 16, 16)] for hc in hists]
                for c, hc in enumerate(hists):
                    hc[pl.ds(b * 16, 16)] = run
                    run = run + hs[c]
                return run

            lax.fori_loop(0, 256, sbody, zero16)

            def pbody(i, _, src=src, dst=dst):
                if src is None:
                    ixs = [full(i + c * CH) for c in range(NCH)]
                    kvs = [keys[pl.ds((i + c * CH) * 16, 16)]
                           for c in range(NCH)]
                else:
                    ixs = [src[pl.ds((i + c * CH) * 16, 16)]
                           for c in range(NCH)]
                    kvs = [plsc.load_gather(keys, [ix * 16 + lane])
                           for ix in ixs]
                dds = [digit(kv) * 16 + lane for kv in kvs]
                offs = [plsc.load_gather(hists[c], [dds[c]])
                        for c in range(NCH)]
                for c in range(NCH):
                    if dst is None:
                        plsc.store_scatter(rowbuf, [lane * RB + offs[c]],
                                           ixs[c])
                    else:
                        plsc.store_scatter(dst, [offs[c] * 16 + lane], ixs[c])
                for c in range(NCH):
                    plsc.addupdate_scatter(hists[c], [dds[c]], one16)
                return 0

            lax.fori_loop(0, CH, pbody, 0)

        copies = [
            pltpu.async_copy(rowbuf.at[pl.ds(r * RB, S)],
                             out_hbm.at[q0 + r], sem)
            for r in range(16)
        ]
        for c in copies:
            c.wait()
        return 0

    lax.fori_loop(0, NGRP, group, 0)


def _argsort_desc(scores_bits):
    f = pl.kernel(
        _argsort_body,
        out_type=jax.ShapeDtypeStruct((S, S), jnp.int32),
        mesh=plsc.VectorSubcoreMesh(core_axis_name="c", subcore_axis_name="s"),
        scratch_types=[
            pltpu.VMEM((16 * S,), jnp.int32),
            pltpu.VMEM((16 * S,), jnp.int32),
            pltpu.VMEM((16 * S,), jnp.int32),
            pltpu.VMEM((256 * 16,), jnp.int32),
            pltpu.VMEM((256 * 16,), jnp.int32),
            pltpu.VMEM((256 * 16,), jnp.int32),
            pltpu.VMEM((256 * 16,), jnp.int32),
            pltpu.SemaphoreType.DMA,
        ],
        compiler_params=pltpu.CompilerParams(needs_layout_passes=False),
    )
    return f(scores_bits)


def kernel(x, q_resid, freqs_cis, Wq_b, Wk, k_norm_weight, k_norm_bias, Wweights):
    softmax_scale = D ** (-0.5)
    q = (q_resid @ Wq_b.T).reshape(B, S, H, D)
    q_nope, q_pe = q[..., :NOPE], q[..., NOPE:]
    k = _layer_norm(x @ Wk.T, k_norm_weight, k_norm_bias)
    k_nope, k_pe = k[..., :NOPE], k[..., NOPE:]
    q_pe = _apply_rope(q_pe, freqs_cis)
    k_pe = _apply_rope(k_pe, freqs_cis)
    q = jnp.concatenate([q_nope, q_pe], axis=-1)
    k = jnp.concatenate([k_nope, k_pe], axis=-1)
    q = _rotate_activation(q)
    k = _rotate_activation(k)
    weights = (x @ Wweights.T).astype(jnp.float32) * (H ** (-0.5)) * softmax_scale

    qf_h = jnp.transpose(q[0], (1, 0, 2))
    kf_t = jnp.transpose(k[0], (1, 0))
    w_h = jnp.transpose(weights[0], (1, 0))[:, None, :]

    sc = _scores(qf_h, kf_t, w_h)
    scores_bits = lax.bitcast_convert_type(sc, jnp.int32)
    topk_indices = _argsort_desc(scores_bits)
    return topk_indices[None]


# speedup vs baseline: 2.8253x; 1.4647x over previous
import functools

import jax
import jax.numpy as jnp
from jax import lax
from jax.experimental import pallas as pl
from jax.experimental.pallas import tpu as pltpu
from jax.experimental.pallas import tpu_sc as plsc

B, S, HID = 1, 2048, 2048
H, D, ROPE, NOPE, QLORA, TOPK = 16, 128, 64, 64, 1536, 2048


def _hadamard_transform(x, scale):
    shp = x.shape
    n = shp[-1]
    y = x.reshape(-1, n)
    h = 1
    while h < n:
        y = y.reshape(-1, n // (2 * h), 2, h)
        a = y[:, :, 0, :]
        b = y[:, :, 1, :]
        y = jnp.stack([a + b, a - b], axis=2)
        y = y.reshape(-1, n)
        h *= 2
    return (y * scale).reshape(shp)


def _rotate_activation(x):
    xb = x.astype(jnp.bfloat16)
    return _hadamard_transform(xb, xb.shape[-1] ** (-0.5))


def _apply_rope(x, angles):
    cos = jnp.cos(angles)
    sin = jnp.sin(angles)
    if x.ndim == 4:
        cos = cos[None, :, None, :]
        sin = sin[None, :, None, :]
    else:
        cos = cos[None, :, :]
        sin = sin[None, :, :]
    xr = x[..., 0::2].astype(jnp.float32)
    xi = x[..., 1::2].astype(jnp.float32)
    yr = xr * cos - xi * sin
    yi = xr * sin + xi * cos
    y = jnp.stack([yr, yi], axis=-1).reshape(x.shape)
    return y.astype(x.dtype)


def _layer_norm(x, g, b, eps=1e-5):
    m = jnp.mean(x, axis=-1, keepdims=True)
    v = jnp.var(x, axis=-1, keepdims=True)
    return (x - m) / jnp.sqrt(v + eps) * g + b


BQ = 512


def _scores_kernel(qf_ref, kf_ref, w_ref, out_ref):
    h = pl.program_id(1)
    s = lax.dot_general(qf_ref[...], kf_ref[...],
                        (((1,), (1,)), ((), ())),
                        preferred_element_type=jnp.float32)
    s = jnp.maximum(s, 0.0) * w_ref[0, 0][:, None]

    @pl.when(h == 0)
    def _():
        out_ref[...] = s

    @pl.when(h > 0)
    def _():
        out_ref[...] += s


def _scores(qf, kf, w_h):
    return pl.pallas_call(
        _scores_kernel,
        grid=(S // BQ, H),
        in_specs=[
            pl.BlockSpec((BQ, D), lambda i, h: (i, h)),
            pl.BlockSpec((S, D), lambda i, h: (0, 0)),
            pl.BlockSpec((1, 1, BQ), lambda i, h: (h, 0, i)),
        ],
        out_specs=pl.BlockSpec((BQ, S), lambda i, h: (i, 0)),
        out_shape=jax.ShapeDtypeStruct((S, S), jnp.float32),
    )(qf, kf, w_h)


def _hadamard_kernel(x_ref, p_ref, g_ref, out_ref):
    yb = x_ref[...].astype(jnp.bfloat16)
    for st in range(7):
        perm = lax.dot_general(yb, p_ref[st], (((1,), (0,)), ((), ())),
                               preferred_element_type=jnp.float32)
        yb = (yb * g_ref[st][None, :] + perm).astype(jnp.bfloat16)
    out_ref[...] = yb * (D ** (-0.5))


def _hadamard_q(q2):
    import numpy as np
    j = np.arange(D)
    pmats = np.zeros((7, D, D), np.float32)
    sgns = np.zeros((7, D), np.float32)
    for st in range(7):
        h = 1 << st
        pmats[st, j ^ h, j] = 1.0
        sgns[st] = np.where(j & h, -1.0, 1.0)
    p = jnp.asarray(pmats, dtype=jnp.bfloat16)
    g = jnp.asarray(sgns, dtype=jnp.bfloat16)
    return pl.pallas_call(
        _hadamard_kernel,
        grid=(S // BQ, H),
        in_specs=[
            pl.BlockSpec((BQ, D), lambda i, h: (i, h)),
            pl.BlockSpec((7, D, D), lambda i, h: (0, 0, 0)),
            pl.BlockSpec((7, D), lambda i, h: (0, 0)),
        ],
        out_specs=pl.BlockSpec((BQ, D), lambda i, h: (i, h)),
        out_shape=jax.ShapeDtypeStruct((S, H * D), jnp.bfloat16),
    )(q2, p, g)


NW = 32
RPT = S // NW
NGRP = RPT // 16


NCH = 4
CH = S // NCH
RB = S


def _argsort_body(scores_hbm, out_hbm, rowbuf, keys, ia, h0, h1, h2, h3, sem):
    lane = lax.iota(jnp.int32, 16)
    zero16 = jnp.zeros((16,), jnp.int32)
    one16 = jnp.ones((16,), jnp.int32)
    hists = [h0, h1, h2, h3]

    def full(v):
        return jnp.full((16,), v, jnp.int32)

    wid = lax.axis_index("s") * 2 + lax.axis_index("c")

    def group(g, _):
        q0 = wid * RPT + g * 16

        copies = [
            pltpu.async_copy(scores_hbm.at[q0 + r],
                             rowbuf.at[pl.ds(r * RB, S)], sem)
            for r in range(16)
        ]
        for c in copies:
            c.wait()

        def tbody(i, _):
            jvs = [jnp.bitwise_and(full(i + c * CH) + lane, S - 1)
                   for c in range(NCH)]
            us = [plsc.load_gather(rowbuf, [lane * RB + jv]) for jv in jvs]
            ks = []
            for u in us:
                m = lax.shift_right_arithmetic(u, 31)
                xorv = jnp.bitwise_xor(
                    jnp.bitwise_or(m, jnp.int32(-2147483648)), jnp.int32(-1))
                ks.append(jnp.bitwise_xor(u, xorv))
            for jv, kv in zip(jvs, ks):
                plsc.store_scatter(keys, [jv * 16 + lane], kv)
            return 0

        lax.fori_loop(0, CH, tbody, 0)

        for p, (src, dst) in enumerate(
                [(None, ia), (ia, rowbuf), (rowbuf, ia), (ia, None)]):
            sh = 8 * p

            def digit(kv, sh=sh):
                d = lax.shift_right_logical(kv, sh) if sh else kv
                return jnp.bitwise_and(d, 255) if sh < 24 else d

            def zbody(b, _):
                for hc in hists:
                    hc[pl.ds(b * 16, 16)] = zero16
                return 0

            lax.fori_loop(0, 256, zbody, 0)

            def hbody(i, _, src=src):
                if src is None:
                    kvs = [keys[pl.ds((i + c * CH) * 16, 16)]
                           for c in range(NCH)]
                else:
                    ixs = [src[pl.ds((i + c * CH) * 16, 16)]
                           for c in range(NCH)]
                    kvs = [plsc.load_gather(keys, [ix * 16 + lane])
                           for ix in ixs]
                dds = [digit(kv) * 16 + lane for kv in kvs]
                for c in range(NCH):
                    plsc.addupdate_scatter(hists[c], [dds[c]], one16)
                return 0

            lax.fori_loop(0, CH, hbody, 0)

            def sbody(b, run):
                hs = [hc[pl.ds(b * 16, 16)] for hc in hists]
                for c, hc in enumerate(hists):
                    hc[pl.ds(b * 16, 16)] = run
                    run = run + hs[c]
                return run

            lax.fori_loop(0, 256, sbody, zero16)

            def pbody(i, _, src=src, dst=dst):
                if src is None:
                    ixs = [full(i + c * CH) for c in range(NCH)]
                    kvs = [keys[pl.ds((i + c * CH) * 16, 16)]
                           for c in range(NCH)]
                else:
                    ixs = [src[pl.ds((i + c * CH) * 16, 16)]
                           for c in range(NCH)]
                    kvs = [plsc.load_gather(keys, [ix * 16 + lane])
                           for ix in ixs]
                dds = [digit(kv) * 16 + lane for kv in kvs]
                offs = [plsc.load_gather(hists[c], [dds[c]])
                        for c in range(NCH)]
                for c in range(NCH):
                    if dst is None:
                        plsc.store_scatter(rowbuf, [lane * RB + offs[c]],
                                           ixs[c])
                    else:
                        plsc.store_scatter(dst, [offs[c] * 16 + lane], ixs[c])
                for c in range(NCH):
                    plsc.addupdate_scatter(hists[c], [dds[c]], one16)
                return 0

            lax.fori_loop(0, CH, pbody, 0)

        copies = [
            pltpu.async_copy(rowbuf.at[pl.ds(r * RB, S)],
                             out_hbm.at[q0 + r], sem)
            for r in range(16)
        ]
        for c in copies:
            c.wait()
        return 0

    lax.fori_loop(0, NGRP, group, 0)


def _argsort_desc(scores_bits):
    f = pl.kernel(
        _argsort_body,
        out_type=jax.ShapeDtypeStruct((S, S), jnp.int32),
        mesh=plsc.VectorSubcoreMesh(core_axis_name="c", subcore_axis_name="s"),
        scratch_types=[
            pltpu.VMEM((16 * S,), jnp.int32),
            pltpu.VMEM((16 * S,), jnp.int32),
            pltpu.VMEM((16 * S,), jnp.int32),
            pltpu.VMEM((256 * 16,), jnp.int32),
            pltpu.VMEM((256 * 16,), jnp.int32),
            pltpu.VMEM((256 * 16,), jnp.int32),
            pltpu.VMEM((256 * 16,), jnp.int32),
            pltpu.SemaphoreType.DMA,
        ],
        compiler_params=pltpu.CompilerParams(needs_layout_passes=False),
    )
    return f(scores_bits)


def kernel(x, q_resid, freqs_cis, Wq_b, Wk, k_norm_weight, k_norm_bias, Wweights):
    softmax_scale = D ** (-0.5)
    q = (q_resid @ Wq_b.T).reshape(B, S, H, D)
    q_nope, q_pe = q[..., :NOPE], q[..., NOPE:]
    k = _layer_norm(x @ Wk.T, k_norm_weight, k_norm_bias)
    k_nope, k_pe = k[..., :NOPE], k[..., NOPE:]
    q_pe = _apply_rope(q_pe, freqs_cis)
    k_pe = _apply_rope(k_pe, freqs_cis)
    q = jnp.concatenate([q_nope, q_pe], axis=-1)
    k = jnp.concatenate([k_nope, k_pe], axis=-1)
    k = _rotate_activation(k)
    weights = (x @ Wweights.T).astype(jnp.float32) * (H ** (-0.5)) * softmax_scale

    qf = _hadamard_q(q.reshape(S, H * D))
    kf = k[0]
    w_h = jnp.transpose(weights[0], (1, 0))[:, None, :]

    sc = _scores(qf, kf, w_h)
    scores_bits = lax.bitcast_convert_type(sc, jnp.int32)
    topk_indices = _argsort_desc(scores_bits)
    return topk_indices[None]


# speedup vs baseline: 3.2403x; 1.1469x over previous
import functools

import jax
import jax.numpy as jnp
from jax import lax
from jax.experimental import pallas as pl
from jax.experimental.pallas import tpu as pltpu
from jax.experimental.pallas import tpu_sc as plsc

B, S, HID = 1, 2048, 2048
H, D, ROPE, NOPE, QLORA, TOPK = 16, 128, 64, 64, 1536, 2048


def _hadamard_transform(x, scale):
    shp = x.shape
    n = shp[-1]
    y = x.reshape(-1, n)
    h = 1
    while h < n:
        y = y.reshape(-1, n // (2 * h), 2, h)
        a = y[:, :, 0, :]
        b = y[:, :, 1, :]
        y = jnp.stack([a + b, a - b], axis=2)
        y = y.reshape(-1, n)
        h *= 2
    return (y * scale).reshape(shp)


def _rotate_activation(x):
    xb = x.astype(jnp.bfloat16)
    return _hadamard_transform(xb, xb.shape[-1] ** (-0.5))


def _apply_rope(x, angles):
    cos = jnp.cos(angles)
    sin = jnp.sin(angles)
    if x.ndim == 4:
        cos = cos[None, :, None, :]
        sin = sin[None, :, None, :]
    else:
        cos = cos[None, :, :]
        sin = sin[None, :, :]
    xr = x[..., 0::2].astype(jnp.float32)
    xi = x[..., 1::2].astype(jnp.float32)
    yr = xr * cos - xi * sin
    yi = xr * sin + xi * cos
    y = jnp.stack([yr, yi], axis=-1).reshape(x.shape)
    return y.astype(x.dtype)


def _layer_norm(x, g, b, eps=1e-5):
    m = jnp.mean(x, axis=-1, keepdims=True)
    v = jnp.var(x, axis=-1, keepdims=True)
    return (x - m) / jnp.sqrt(v + eps) * g + b


BQ = 512


def _scores_kernel(qf_ref, kf_ref, w_ref, out_ref):
    h = pl.program_id(1)
    s = lax.dot_general(qf_ref[...], kf_ref[...],
                        (((1,), (1,)), ((), ())),
                        preferred_element_type=jnp.float32)
    s = jnp.maximum(s, 0.0) * w_ref[0, 0][:, None]

    @pl.when(h == 0)
    def _():
        out_ref[...] = s

    @pl.when(h > 0)
    def _():
        out_ref[...] += s


def _scores(qf, kf, w_h):
    return pl.pallas_call(
        _scores_kernel,
        grid=(S // BQ, H),
        in_specs=[
            pl.BlockSpec((BQ, D), lambda i, h: (i, h)),
            pl.BlockSpec((S, D), lambda i, h: (0, 0)),
            pl.BlockSpec((1, 1, BQ), lambda i, h: (h, 0, i)),
        ],
        out_specs=pl.BlockSpec((BQ, S), lambda i, h: (i, 0)),
        out_shape=jax.ShapeDtypeStruct((S, S), jnp.float32),
    )(qf, kf, w_h)


def _rope_hadamard_kernel(x_ref, c_ref, s_ref, p_ref, g_ref, out_ref):
    x = x_ref[...]
    even = (lax.broadcasted_iota(jnp.int32, x.shape, 1) % 2) == 0
    sw = jnp.where(even, pltpu.roll(x, D - 1, 1), pltpu.roll(x, 1, 1))
    y = x * c_ref[...] + sw * s_ref[...]
    yb = y.astype(jnp.bfloat16)
    for st in range(7):
        perm = lax.dot_general(yb, p_ref[st], (((1,), (0,)), ((), ())),
                               preferred_element_type=jnp.float32)
        yb = (yb * g_ref[st][None, :] + perm).astype(jnp.bfloat16)
    out_ref[...] = yb * (D ** (-0.5))


def _rope_hadamard_q(q2, cpad, spad):
    import numpy as np
    j = np.arange(D)
    pmats = np.zeros((7, D, D), np.float32)
    sgns = np.zeros((7, D), np.float32)
    for st in range(7):
        h = 1 << st
        pmats[st, j ^ h, j] = 1.0
        sgns[st] = np.where(j & h, -1.0, 1.0)
    p = jnp.asarray(pmats, dtype=jnp.bfloat16)
    g = jnp.asarray(sgns, dtype=jnp.bfloat16)
    return pl.pallas_call(
        _rope_hadamard_kernel,
        grid=(S // BQ, H),
        in_specs=[
            pl.BlockSpec((BQ, D), lambda i, h: (i, h)),
            pl.BlockSpec((BQ, D), lambda i, h: (i, 0)),
            pl.BlockSpec((BQ, D), lambda i, h: (i, 0)),
            pl.BlockSpec((7, D, D), lambda i, h: (0, 0, 0)),
            pl.BlockSpec((7, D), lambda i, h: (0, 0)),
        ],
        out_specs=pl.BlockSpec((BQ, D), lambda i, h: (i, h)),
        out_shape=jax.ShapeDtypeStruct((S, H * D), jnp.bfloat16),
    )(q2, cpad, spad, p, g)


NW = 32
RPT = S // NW
NGRP = RPT // 16


NCH = 4
CH = S // NCH
RB = S


def _argsort_body(scores_hbm, out_hbm, rowbuf, keys, ia, h0, h1, h2, h3, sem):
    lane = lax.iota(jnp.int32, 16)
    zero16 = jnp.zeros((16,), jnp.int32)
    one16 = jnp.ones((16,), jnp.int32)
    hists = [h0, h1, h2, h3]

    def full(v):
        return jnp.full((16,), v, jnp.int32)

    wid = lax.axis_index("s") * 2 + lax.axis_index("c")

    def group(g, _):
        q0 = wid * RPT + g * 16

        copies = [
            pltpu.async_copy(scores_hbm.at[q0 + r],
                             rowbuf.at[pl.ds(r * RB, S)], sem)
            for r in range(16)
        ]
        for c in copies:
            c.wait()

        def tbody(i, _):
            jvs = [jnp.bitwise_and(full(i + c * CH) + lane, S - 1)
                   for c in range(NCH)]
            us = [plsc.load_gather(rowbuf, [lane * RB + jv]) for jv in jvs]
            ks = []
            for u in us:
                m = lax.shift_right_arithmetic(u, 31)
                xorv = jnp.bitwise_xor(
                    jnp.bitwise_or(m, jnp.int32(-2147483648)), jnp.int32(-1))
                ks.append(jnp.bitwise_xor(u, xorv))
            for jv, kv in zip(jvs, ks):
                plsc.store_scatter(keys, [jv * 16 + lane], kv)
            return 0

        lax.fori_loop(0, CH, tbody, 0)

        for p, (src, dst) in enumerate(
                [(None, ia), (ia, rowbuf), (rowbuf, ia), (ia, None)]):
            sh = 8 * p

            def digit(kv, sh=sh):
                d = lax.shift_right_logical(kv, sh) if sh else kv
                return jnp.bitwise_and(d, 255) if sh < 24 else d

            def zbody(b, _):
                for hc in hists:
                    hc[pl.ds(b * 16, 16)] = zero16
                return 0

            lax.fori_loop(0, 256, zbody, 0)

            def hbody(i, _, src=src):
                if src is None:
                    kvs = [keys[pl.ds((i + c * CH) * 16, 16)]
                           for c in range(NCH)]
                else:
                    ixs = [src[pl.ds((i + c * CH) * 16, 16)]
                           for c in range(NCH)]
                    kvs = [plsc.load_gather(keys, [ix * 16 + lane])
                           for ix in ixs]
                dds = [digit(kv) * 16 + lane for kv in kvs]
                for c in range(NCH):
                    plsc.addupdate_scatter(hists[c], [dds[c]], one16)
                return 0

            lax.fori_loop(0, CH, hbody, 0)

            def sbody(b, run):
                hs = [hc[pl.ds(b * 16, 16)] for hc in hists]
                for c, hc in enumerate(hists):
                    hc[pl.ds(b * 16, 16)] = run
                    run = run + hs[c]
                return run

            lax.fori_loop(0, 256, sbody, zero16)

            def pbody(i, _, src=src, dst=dst):
                if src is None:
                    ixs = [full(i + c * CH) for c in range(NCH)]
                    kvs = [keys[pl.ds((i + c * CH) * 16, 16)]
                           for c in range(NCH)]
                else:
                    ixs = [src[pl.ds((i + c * CH) * 16, 16)]
                           for c in range(NCH)]
                    kvs = [plsc.load_gather(keys, [ix * 16 + lane])
                           for ix in ixs]
                dds = [digit(kv) * 16 + lane for kv in kvs]
                offs = [plsc.load_gather(hists[c], [dds[c]])
                        for c in range(NCH)]
                for c in range(NCH):
                    if dst is None:
                        plsc.store_scatter(rowbuf, [lane * RB + offs[c]],
                                           ixs[c])
                    else:
                        plsc.store_scatter(dst, [offs[c] * 16 + lane], ixs[c])
                for c in range(NCH):
                    plsc.addupdate_scatter(hists[c], [dds[c]], one16)
                return 0

            lax.fori_loop(0, CH, pbody, 0)

        copies = [
            pltpu.async_copy(rowbuf.at[pl.ds(r * RB, S)],
                             out_hbm.at[q0 + r], sem)
            for r in range(16)
        ]
        for c in copies:
            c.wait()
        return 0

    lax.fori_loop(0, NGRP, group, 0)


def _argsort_desc(scores_bits):
    f = pl.kernel(
        _argsort_body,
        out_type=jax.ShapeDtypeStruct((S, S), jnp.int32),
        mesh=plsc.VectorSubcoreMesh(core_axis_name="c", subcore_axis_name="s"),
        scratch_types=[
            pltpu.VMEM((16 * S,), jnp.int32),
            pltpu.VMEM((16 * S,), jnp.int32),
            pltpu.VMEM((16 * S,), jnp.int32),
            pltpu.VMEM((256 * 16,), jnp.int32),
            pltpu.VMEM((256 * 16,), jnp.int32),
            pltpu.VMEM((256 * 16,), jnp.int32),
            pltpu.VMEM((256 * 16,), jnp.int32),
            pltpu.SemaphoreType.DMA,
        ],
        compiler_params=pltpu.CompilerParams(needs_layout_passes=False),
    )
    return f(scores_bits)


def kernel(x, q_resid, freqs_cis, Wq_b, Wk, k_norm_weight, k_norm_bias, Wweights):
    softmax_scale = D ** (-0.5)
    qraw = (q_resid @ Wq_b.T).reshape(S, H * D)
    k = _layer_norm(x @ Wk.T, k_norm_weight, k_norm_bias)
    k_nope, k_pe = k[..., :NOPE], k[..., NOPE:]
    k_pe = _apply_rope(k_pe, freqs_cis)
    k = jnp.concatenate([k_nope, k_pe], axis=-1)
    k = _rotate_activation(k)
    weights = (x @ Wweights.T).astype(jnp.float32) * (H ** (-0.5)) * softmax_scale

    cos = jnp.cos(freqs_cis)
    sin = jnp.sin(freqs_cis)
    c_rep = jnp.repeat(cos, 2, axis=1)
    s_alt = jnp.stack([-sin, sin], axis=-1).reshape(S, ROPE)
    ones = jnp.ones((S, NOPE), jnp.float32)
    zeros = jnp.zeros((S, NOPE), jnp.float32)
    cpad = jnp.concatenate([ones, c_rep], axis=1)
    spad = jnp.concatenate([zeros, s_alt], axis=1)

    qf = _rope_hadamard_q(qraw, cpad, spad)
    kf = k[0]
    w_h = jnp.transpose(weights[0], (1, 0))[:, None, :]

    sc = _scores(qf, kf, w_h)
    scores_bits = lax.bitcast_convert_type(sc, jnp.int32)
    topk_indices = _argsort_desc(scores_bits)
    return topk_indices[None]


# speedup vs baseline: 3.6983x; 1.1414x over previous
import functools

import jax
import jax.numpy as jnp
from jax import lax
from jax.experimental import pallas as pl
from jax.experimental.pallas import tpu as pltpu
from jax.experimental.pallas import tpu_sc as plsc

B, S, HID = 1, 2048, 2048
H, D, ROPE, NOPE, QLORA, TOPK = 16, 128, 64, 64, 1536, 2048


def _hadamard_transform(x, scale):
    shp = x.shape
    n = shp[-1]
    y = x.reshape(-1, n)
    h = 1
    while h < n:
        y = y.reshape(-1, n // (2 * h), 2, h)
        a = y[:, :, 0, :]
        b = y[:, :, 1, :]
        y = jnp.stack([a + b, a - b], axis=2)
        y = y.reshape(-1, n)
        h *= 2
    return (y * scale).reshape(shp)


def _rotate_activation(x):
    xb = x.astype(jnp.bfloat16)
    return _hadamard_transform(xb, xb.shape[-1] ** (-0.5))


def _apply_rope(x, angles):
    cos = jnp.cos(angles)
    sin = jnp.sin(angles)
    if x.ndim == 4:
        cos = cos[None, :, None, :]
        sin = sin[None, :, None, :]
    else:
        cos = cos[None, :, :]
        sin = sin[None, :, :]
    xr = x[..., 0::2].astype(jnp.float32)
    xi = x[..., 1::2].astype(jnp.float32)
    yr = xr * cos - xi * sin
    yi = xr * sin + xi * cos
    y = jnp.stack([yr, yi], axis=-1).reshape(x.shape)
    return y.astype(x.dtype)


def _layer_norm(x, g, b, eps=1e-5):
    m = jnp.mean(x, axis=-1, keepdims=True)
    v = jnp.var(x, axis=-1, keepdims=True)
    return (x - m) / jnp.sqrt(v + eps) * g + b


BQ = 512


def _scores_kernel(qf_ref, kf_ref, w_ref, out_ref):
    h = pl.program_id(1)
    s = lax.dot_general(qf_ref[...], kf_ref[...],
                        (((1,), (1,)), ((), ())),
                        preferred_element_type=jnp.float32)
    s = jnp.maximum(s, 0.0) * w_ref[0, 0][:, None]

    @pl.when(h == 0)
    def _():
        out_ref[...] = s

    @pl.when(h > 0)
    def _():
        out_ref[...] += s


def _scores(qf, kf, w_h):
    return pl.pallas_call(
        _scores_kernel,
        grid=(S // BQ, H),
        in_specs=[
            pl.BlockSpec((BQ, D), lambda i, h: (i, h)),
            pl.BlockSpec((S, D), lambda i, h: (0, 0)),
            pl.BlockSpec((1, 1, BQ), lambda i, h: (h, 0, i)),
        ],
        out_specs=pl.BlockSpec((BQ, S), lambda i, h: (i, 0)),
        out_shape=jax.ShapeDtypeStruct((S, S), jnp.float32),
    )(qf, kf, w_h)


def _rope_hadamard_kernel(x_ref, c_ref, s_ref, p_ref, g_ref, out_ref):
    x = x_ref[...]
    even = (lax.broadcasted_iota(jnp.int32, x.shape, 1) % 2) == 0
    sw = jnp.where(even, pltpu.roll(x, D - 1, 1), pltpu.roll(x, 1, 1))
    y = x * c_ref[...] + sw * s_ref[...]
    yb = y.astype(jnp.bfloat16)
    for st in range(7):
        perm = lax.dot_general(yb, p_ref[st], (((1,), (0,)), ((), ())),
                               preferred_element_type=jnp.float32)
        yb = (yb * g_ref[st][None, :] + perm).astype(jnp.bfloat16)
    out_ref[...] = yb * (D ** (-0.5))


def _rope_hadamard_q(q2, cpad, spad):
    import numpy as np
    j = np.arange(D)
    pmats = np.zeros((7, D, D), np.float32)
    sgns = np.zeros((7, D), np.float32)
    for st in range(7):
        h = 1 << st
        pmats[st, j ^ h, j] = 1.0
        sgns[st] = np.where(j & h, -1.0, 1.0)
    p = jnp.asarray(pmats, dtype=jnp.bfloat16)
    g = jnp.asarray(sgns, dtype=jnp.bfloat16)
    return pl.pallas_call(
        _rope_hadamard_kernel,
        grid=(S // BQ, H),
        in_specs=[
            pl.BlockSpec((BQ, D), lambda i, h: (i, h)),
            pl.BlockSpec((BQ, D), lambda i, h: (i, 0)),
            pl.BlockSpec((BQ, D), lambda i, h: (i, 0)),
            pl.BlockSpec((7, D, D), lambda i, h: (0, 0, 0)),
            pl.BlockSpec((7, D), lambda i, h: (0, 0)),
        ],
        out_specs=pl.BlockSpec((BQ, D), lambda i, h: (i, h)),
        out_shape=jax.ShapeDtypeStruct((S, H * D), jnp.bfloat16),
    )(q2, cpad, spad, p, g)


NW = 32
RPT = S // NW
NGRP = RPT // 16


NCH = 4
CH = S // NCH
RB = S


def _argsort_body(scores_hbm, out_hbm, rowbuf, keys, ia, h0, h1, h2, h3, sem):
    lane = lax.iota(jnp.int32, 16)
    zero16 = jnp.zeros((16,), jnp.int32)
    one16 = jnp.ones((16,), jnp.int32)
    hists = [h0, h1, h2, h3]

    def full(v):
        return jnp.full((16,), v, jnp.int32)

    wid = lax.axis_index("s") * 2 + lax.axis_index("c")

    def group(g, _):
        q0 = wid * RPT + g * 16

        copies = [
            pltpu.async_copy(scores_hbm.at[q0 + r],
                             rowbuf.at[pl.ds(r * RB, S)], sem)
            for r in range(16)
        ]
        for c in copies:
            c.wait()

        def zero_hists():
            def zbody(b, _):
                for hc in hists:
                    hc[pl.ds(b * 16, 16)] = zero16
                return 0

            lax.fori_loop(0, 256, zbody, 0)

        def scan_hists():
            def sbody(b, run):
                hs = [hc[pl.ds(b * 16, 16)] for hc in hists]
                for c, hc in enumerate(hists):
                    hc[pl.ds(b * 16, 16)] = run
                    run = run + hs[c]
                return run

            lax.fori_loop(0, 256, sbody, zero16)

        zero_hists()

        def tbody(i, _):
            jvs = [full(c * CH) + jnp.bitwise_and(full(i) + lane, CH - 1)
                   for c in range(NCH)]
            us = [plsc.load_gather(rowbuf, [lane * RB + jv]) for jv in jvs]
            ks = []
            for u in us:
                m = lax.shift_right_arithmetic(u, 31)
                xorv = jnp.bitwise_xor(
                    jnp.bitwise_or(m, jnp.int32(-2147483648)), jnp.int32(-1))
                ks.append(jnp.bitwise_xor(u, xorv))
            for jv, kv in zip(jvs, ks):
                plsc.store_scatter(keys, [jv * 16 + lane], kv)
            for c in range(NCH):
                dd = jnp.bitwise_and(ks[c], 255) * 16 + lane
                plsc.addupdate_scatter(hists[c], [dd], one16)
            return 0

        lax.fori_loop(0, CH, tbody, 0)
        scan_hists()

        def p0body(i, _):
            kvs = [keys[pl.ds((i + c * CH) * 16, 16)] for c in range(NCH)]
            dds = [jnp.bitwise_and(kv, 255) * 16 + lane for kv in kvs]
            offs = [plsc.load_gather(hists[c], [dds[c]]) for c in range(NCH)]
            vals = [jnp.bitwise_or(
                        full(i + c * CH),
                        lax.shift_left(jnp.bitwise_and(kvs[c], 0xFF00), 8))
                    for c in range(NCH)]
            for c in range(NCH):
                plsc.store_scatter(ia, [offs[c] * 16 + lane], vals[c])
            for c in range(NCH):
                plsc.addupdate_scatter(hists[c], [dds[c]], one16)
            return 0

        lax.fori_loop(0, CH, p0body, 0)

        for p, (src, dst) in enumerate(
                [(ia, rowbuf), (rowbuf, ia), (ia, None)], start=1):
            zero_hists()

            def hbody(i, _, src=src):
                vals = [src[pl.ds((i + c * CH) * 16, 16)] for c in range(NCH)]
                dds = [lax.shift_right_logical(v, 16) * 16 + lane
                       for v in vals]
                for c in range(NCH):
                    plsc.addupdate_scatter(hists[c], [dds[c]], one16)
                return 0

            lax.fori_loop(0, CH, hbody, 0)
            scan_hists()

            def pbody(i, _, p=p, src=src, dst=dst):
                vals = [src[pl.ds((i + c * CH) * 16, 16)] for c in range(NCH)]
                dds = [lax.shift_right_logical(v, 16) * 16 + lane
                       for v in vals]
                ixs = [jnp.bitwise_and(v, S - 1) for v in vals]
                offs = [plsc.load_gather(hists[c], [dds[c]])
                        for c in range(NCH)]
                if dst is None:
                    for c in range(NCH):
                        plsc.store_scatter(rowbuf, [lane * RB + offs[c]],
                                           ixs[c])
                else:
                    kvs = [plsc.load_gather(keys, [ix * 16 + lane])
                           for ix in ixs]
                    sh = 0 if p == 1 else 8
                    nvals = [jnp.bitwise_or(
                                 ixs[c],
                                 jnp.bitwise_and(
                                     lax.shift_right_logical(kvs[c], sh),
                                     0xFF0000))
                             for c in range(NCH)]
                    for c in range(NCH):
                        plsc.store_scatter(dst, [offs[c] * 16 + lane],
                                           nvals[c])
                for c in range(NCH):
                    plsc.addupdate_scatter(hists[c], [dds[c]], one16)
                return 0

            lax.fori_loop(0, CH, pbody, 0)

        copies = [
            pltpu.async_copy(rowbuf.at[pl.ds(r * RB, S)],
                             out_hbm.at[q0 + r], sem)
            for r in range(16)
        ]
        for c in copies:
            c.wait()
        return 0

    lax.fori_loop(0, NGRP, group, 0)


def _argsort_desc(scores_bits):
    f = pl.kernel(
        _argsort_body,
        out_type=jax.ShapeDtypeStruct((S, S), jnp.int32),
        mesh=plsc.VectorSubcoreMesh(core_axis_name="c", subcore_axis_name="s"),
        scratch_types=[
            pltpu.VMEM((16 * S,), jnp.int32),
            pltpu.VMEM((16 * S,), jnp.int32),
            pltpu.VMEM((16 * S,), jnp.int32),
            pltpu.VMEM((256 * 16,), jnp.int32),
            pltpu.VMEM((256 * 16,), jnp.int32),
            pltpu.VMEM((256 * 16,), jnp.int32),
            pltpu.VMEM((256 * 16,), jnp.int32),
            pltpu.SemaphoreType.DMA,
        ],
        compiler_params=pltpu.CompilerParams(needs_layout_passes=False),
    )
    return f(scores_bits)


def kernel(x, q_resid, freqs_cis, Wq_b, Wk, k_norm_weight, k_norm_bias, Wweights):
    softmax_scale = D ** (-0.5)
    qraw = (q_resid @ Wq_b.T).reshape(S, H * D)
    k = _layer_norm(x @ Wk.T, k_norm_weight, k_norm_bias)
    k_nope, k_pe = k[..., :NOPE], k[..., NOPE:]
    k_pe = _apply_rope(k_pe, freqs_cis)
    k = jnp.concatenate([k_nope, k_pe], axis=-1)
    k = _rotate_activation(k)
    weights = (x @ Wweights.T).astype(jnp.float32) * (H ** (-0.5)) * softmax_scale

    cos = jnp.cos(freqs_cis)
    sin = jnp.sin(freqs_cis)
    c_rep = jnp.repeat(cos, 2, axis=1)
    s_alt = jnp.stack([-sin, sin], axis=-1).reshape(S, ROPE)
    ones = jnp.ones((S, NOPE), jnp.float32)
    zeros = jnp.zeros((S, NOPE), jnp.float32)
    cpad = jnp.concatenate([ones, c_rep], axis=1)
    spad = jnp.concatenate([zeros, s_alt], axis=1)

    qf = _rope_hadamard_q(qraw, cpad, spad)
    kf = k[0]
    w_h = jnp.transpose(weights[0], (1, 0))[:, None, :]

    sc = _scores(qf, kf, w_h)
    scores_bits = lax.bitcast_convert_type(sc, jnp.int32)
    topk_indices = _argsort_desc(scores_bits)
    return topk_indices[None]


# speedup vs baseline: 3.8352x; 1.0370x over previous
import functools

import jax
import jax.numpy as jnp
from jax import lax
from jax.experimental import pallas as pl
from jax.experimental.pallas import tpu as pltpu
from jax.experimental.pallas import tpu_sc as plsc

B, S, HID = 1, 2048, 2048
H, D, ROPE, NOPE, QLORA, TOPK = 16, 128, 64, 64, 1536, 2048


def _hadamard_transform(x, scale):
    shp = x.shape
    n = shp[-1]
    y = x.reshape(-1, n)
    h = 1
    while h < n:
        y = y.reshape(-1, n // (2 * h), 2, h)
        a = y[:, :, 0, :]
        b = y[:, :, 1, :]
        y = jnp.stack([a + b, a - b], axis=2)
        y = y.reshape(-1, n)
        h *= 2
    return (y * scale).reshape(shp)


def _rotate_activation(x):
    xb = x.astype(jnp.bfloat16)
    return _hadamard_transform(xb, xb.shape[-1] ** (-0.5))


def _apply_rope(x, angles):
    cos = jnp.cos(angles)
    sin = jnp.sin(angles)
    if x.ndim == 4:
        cos = cos[None, :, None, :]
        sin = sin[None, :, None, :]
    else:
        cos = cos[None, :, :]
        sin = sin[None, :, :]
    xr = x[..., 0::2].astype(jnp.float32)
    xi = x[..., 1::2].astype(jnp.float32)
    yr = xr * cos - xi * sin
    yi = xr * sin + xi * cos
    y = jnp.stack([yr, yi], axis=-1).reshape(x.shape)
    return y.astype(x.dtype)


def _layer_norm(x, g, b, eps=1e-5):
    m = jnp.mean(x, axis=-1, keepdims=True)
    v = jnp.var(x, axis=-1, keepdims=True)
    return (x - m) / jnp.sqrt(v + eps) * g + b


BQ = 512


def _butterfly_consts():
    import numpy as np
    j = np.arange(D)
    pmats = np.zeros((7, D, D), np.float32)
    sgns = np.zeros((7, D), np.float32)
    for st in range(7):
        h = 1 << st
        pmats[st, j ^ h, j] = 1.0
        sgns[st] = np.where(j & h, -1.0, 1.0)
    return (jnp.asarray(pmats, dtype=jnp.bfloat16),
            jnp.asarray(sgns, dtype=jnp.bfloat16))


def _rope_hadamard(x, c, s, p_ref, g_ref):
    even = (lax.broadcasted_iota(jnp.int32, x.shape, 1) % 2) == 0
    sw = jnp.where(even, pltpu.roll(x, D - 1, 1), pltpu.roll(x, 1, 1))
    y = x * c + sw * s
    yb = y.astype(jnp.bfloat16)
    for st in range(7):
        perm = lax.dot_general(yb, p_ref[st], (((1,), (0,)), ((), ())),
                               preferred_element_type=jnp.float32)
        yb = (yb * g_ref[st][None, :] + perm).astype(jnp.bfloat16)
    return yb * (D ** (-0.5))


def _scores_kernel(q_ref, c_ref, s_ref, p_ref, g_ref, kf_ref, w_ref, out_ref):
    h = pl.program_id(1)
    qf = _rope_hadamard(q_ref[...], c_ref[...], s_ref[...], p_ref, g_ref)
    s = lax.dot_general(qf, kf_ref[...],
                        (((1,), (1,)), ((), ())),
                        preferred_element_type=jnp.float32)
    s = jnp.maximum(s, 0.0) * w_ref[0, 0][:, None]

    @pl.when(h == 0)
    def _():
        out_ref[...] = s

    @pl.when(h > 0)
    def _():
        out_ref[...] += s


def _scores(qraw, cpad, spad, kf, w_h):
    p, g = _butterfly_consts()
    return pl.pallas_call(
        _scores_kernel,
        grid=(S // BQ, H),
        in_specs=[
            pl.BlockSpec((BQ, D), lambda i, h: (i, h)),
            pl.BlockSpec((BQ, D), lambda i, h: (i, 0)),
            pl.BlockSpec((BQ, D), lambda i, h: (i, 0)),
            pl.BlockSpec((7, D, D), lambda i, h: (0, 0, 0)),
            pl.BlockSpec((7, D), lambda i, h: (0, 0)),
            pl.BlockSpec((S, D), lambda i, h: (0, 0)),
            pl.BlockSpec((1, 1, BQ), lambda i, h: (h, 0, i)),
        ],
        out_specs=pl.BlockSpec((BQ, S), lambda i, h: (i, 0)),
        out_shape=jax.ShapeDtypeStruct((S, S), jnp.float32),
    )(qraw, cpad, spad, p, g, kf, w_h)


def _prep_kernel(x_ref, c_ref, s_ref, p_ref, g_ref, out_ref):
    out_ref[...] = _rope_hadamard(x_ref[...], c_ref[...], s_ref[...],
                                  p_ref, g_ref)


def _prep_k(k_ln, cpad, spad):
    p, g = _butterfly_consts()
    return pl.pallas_call(
        _prep_kernel,
        grid=(S // BQ,),
        in_specs=[
            pl.BlockSpec((BQ, D), lambda i: (i, 0)),
            pl.BlockSpec((BQ, D), lambda i: (i, 0)),
            pl.BlockSpec((BQ, D), lambda i: (i, 0)),
            pl.BlockSpec((7, D, D), lambda i: (0, 0, 0)),
            pl.BlockSpec((7, D), lambda i: (0, 0)),
        ],
        out_specs=pl.BlockSpec((BQ, D), lambda i: (i, 0)),
        out_shape=jax.ShapeDtypeStruct((S, D), jnp.bfloat16),
    )(k_ln, cpad, spad, p, g)


NW = 32
RPT = S // NW
NGRP = RPT // 16


NCH = 4
CH = S // NCH
RB = S


def _argsort_body(scores_hbm, out_hbm, rowbuf, keys, ia, h0, h1, h2, h3, sem):
    lane = lax.iota(jnp.int32, 16)
    zero16 = jnp.zeros((16,), jnp.int32)
    one16 = jnp.ones((16,), jnp.int32)
    hists = [h0, h1, h2, h3]

    def full(v):
        return jnp.full((16,), v, jnp.int32)

    wid = lax.axis_index("s") * 2 + lax.axis_index("c")

    def group(g, _):
        q0 = wid * RPT + g * 16

        copies = [
            pltpu.async_copy(scores_hbm.at[q0 + r],
                             rowbuf.at[pl.ds(r * RB, S)], sem)
            for r in range(16)
        ]
        for c in copies:
            c.wait()

        def zero_hists():
            def zbody(b, _):
                for hc in hists:
                    hc[pl.ds(b * 16, 16)] = zero16
                return 0

            lax.fori_loop(0, 256, zbody, 0)

        def scan_hists():
            def sbody(b, run):
                hs = [hc[pl.ds(b * 16, 16)] for hc in hists]
                for c, hc in enumerate(hists):
                    hc[pl.ds(b * 16, 16)] = run
                    run = run + hs[c]
                return run

            lax.fori_loop(0, 256, sbody, zero16)

        zero_hists()

        def tbody(i, _):
            jvs = [full(c * CH) + jnp.bitwise_and(full(i) + lane, CH - 1)
                   for c in range(NCH)]
            us = [plsc.load_gather(rowbuf, [lane * RB + jv]) for jv in jvs]
            ks = []
            for u in us:
                m = lax.shift_right_arithmetic(u, 31)
                xorv = jnp.bitwise_xor(
                    jnp.bitwise_or(m, jnp.int32(-2147483648)), jnp.int32(-1))
                ks.append(jnp.bitwise_xor(u, xorv))
            for jv, kv in zip(jvs, ks):
                plsc.store_scatter(keys, [jv * 16 + lane], kv)
            for c in range(NCH):
                dd = jnp.bitwise_and(ks[c], 255) * 16 + lane
                plsc.addupdate_scatter(hists[c], [dd], one16)
            return 0

        lax.fori_loop(0, CH, tbody, 0)
        scan_hists()

        def p0body(i, _):
            kvs = [keys[pl.ds((i + c * CH) * 16, 16)] for c in range(NCH)]
            dds = [jnp.bitwise_and(kv, 255) * 16 + lane for kv in kvs]
            offs = [plsc.load_gather(hists[c], [dds[c]]) for c in range(NCH)]
            vals = [jnp.bitwise_or(
                        full(i + c * CH),
                        lax.shift_left(jnp.bitwise_and(kvs[c], 0xFF00), 8))
                    for c in range(NCH)]
            for c in range(NCH):
                plsc.store_scatter(ia, [offs[c] * 16 + lane], vals[c])
            for c in range(NCH):
                plsc.addupdate_scatter(hists[c], [dds[c]], one16)
            return 0

        lax.fori_loop(0, CH, p0body, 0)

        for p, (src, dst) in enumerate(
                [(ia, rowbuf), (rowbuf, ia), (ia, None)], start=1):
            zero_hists()

            def hbody(i, _, src=src):
                vals = [src[pl.ds((i + c * CH) * 16, 16)] for c in range(NCH)]
                dds = [lax.shift_right_logical(v, 16) * 16 + lane
                       for v in vals]
                for c in range(NCH):
                    plsc.addupdate_scatter(hists[c], [dds[c]], one16)
                return 0

            lax.fori_loop(0, CH, hbody, 0)
            scan_hists()

            def pbody(i, _, p=p, src=src, dst=dst):
                vals = [src[pl.ds((i + c * CH) * 16, 16)] for c in range(NCH)]
                dds = [lax.shift_right_logical(v, 16) * 16 + lane
                       for v in vals]
                ixs = [jnp.bitwise_and(v, S - 1) for v in vals]
                offs = [plsc.load_gather(hists[c], [dds[c]])
                        for c in range(NCH)]
                if dst is None:
                    for c in range(NCH):
                        plsc.store_scatter(rowbuf, [lane * RB + offs[c]],
                                           ixs[c])
                else:
                    kvs = [plsc.load_gather(keys, [ix * 16 + lane])
                           for ix in ixs]
                    sh = 0 if p == 1 else 8
                    nvals = [jnp.bitwise_or(
                                 ixs[c],
                                 jnp.bitwise_and(
                                     lax.shift_right_logical(kvs[c], sh),
                                     0xFF0000))
                             for c in range(NCH)]
                    for c in range(NCH):
                        plsc.store_scatter(dst, [offs[c] * 16 + lane],
                                           nvals[c])
                for c in range(NCH):
                    plsc.addupdate_scatter(hists[c], [dds[c]], one16)
                return 0

            lax.fori_loop(0, CH, pbody, 0)

        copies = [
            pltpu.async_copy(rowbuf.at[pl.ds(r * RB, S)],
                             out_hbm.at[q0 + r], sem)
            for r in range(16)
        ]
        for c in copies:
            c.wait()
        return 0

    lax.fori_loop(0, NGRP, group, 0)


def _argsort_desc(scores_bits):
    f = pl.kernel(
        _argsort_body,
        out_type=jax.ShapeDtypeStruct((S, S), jnp.int32),
        mesh=plsc.VectorSubcoreMesh(core_axis_name="c", subcore_axis_name="s"),
        scratch_types=[
            pltpu.VMEM((16 * S,), jnp.int32),
            pltpu.VMEM((16 * S,), jnp.int32),
            pltpu.VMEM((16 * S,), jnp.int32),
            pltpu.VMEM((256 * 16,), jnp.int32),
            pltpu.VMEM((256 * 16,), jnp.int32),
            pltpu.VMEM((256 * 16,), jnp.int32),
            pltpu.VMEM((256 * 16,), jnp.int32),
            pltpu.SemaphoreType.DMA,
        ],
        compiler_params=pltpu.CompilerParams(needs_layout_passes=False),
    )
    return f(scores_bits)


def kernel(x, q_resid, freqs_cis, Wq_b, Wk, k_norm_weight, k_norm_bias, Wweights):
    softmax_scale = D ** (-0.5)
    qraw = (q_resid @ Wq_b.T).reshape(S, H * D)
    k_ln = _layer_norm(x @ Wk.T, k_norm_weight, k_norm_bias)[0]
    weights = (x @ Wweights.T).astype(jnp.float32) * (H ** (-0.5)) * softmax_scale

    cos = jnp.cos(freqs_cis)
    sin = jnp.sin(freqs_cis)
    c_rep = jnp.repeat(cos, 2, axis=1)
    s_alt = jnp.stack([-sin, sin], axis=-1).reshape(S, ROPE)
    ones = jnp.ones((S, NOPE), jnp.float32)
    zeros = jnp.zeros((S, NOPE), jnp.float32)
    cpad = jnp.concatenate([ones, c_rep], axis=1)
    spad = jnp.concatenate([zeros, s_alt], axis=1)

    kf = _prep_k(k_ln, cpad, spad)
    w_h = jnp.transpose(weights[0], (1, 0))[:, None, :]

    sc = _scores(qraw, cpad, spad, kf, w_h)
    scores_bits = lax.bitcast_convert_type(sc, jnp.int32)
    topk_indices = _argsort_desc(scores_bits)
    return topk_indices[None]


# speedup vs baseline: 4.2102x; 1.0978x over previous
import functools

import jax
import jax.numpy as jnp
from jax import lax
from jax.experimental import pallas as pl
from jax.experimental.pallas import tpu as pltpu
from jax.experimental.pallas import tpu_sc as plsc

B, S, HID = 1, 2048, 2048
H, D, ROPE, NOPE, QLORA, TOPK = 16, 128, 64, 64, 1536, 2048


def _hadamard_transform(x, scale):
    shp = x.shape
    n = shp[-1]
    y = x.reshape(-1, n)
    h = 1
    while h < n:
        y = y.reshape(-1, n // (2 * h), 2, h)
        a = y[:, :, 0, :]
        b = y[:, :, 1, :]
        y = jnp.stack([a + b, a - b], axis=2)
        y = y.reshape(-1, n)
        h *= 2
    return (y * scale).reshape(shp)


def _rotate_activation(x):
    xb = x.astype(jnp.bfloat16)
    return _hadamard_transform(xb, xb.shape[-1] ** (-0.5))


def _apply_rope(x, angles):
    cos = jnp.cos(angles)
    sin = jnp.sin(angles)
    if x.ndim == 4:
        cos = cos[None, :, None, :]
        sin = sin[None, :, None, :]
    else:
        cos = cos[None, :, :]
        sin = sin[None, :, :]
    xr = x[..., 0::2].astype(jnp.float32)
    xi = x[..., 1::2].astype(jnp.float32)
    yr = xr * cos - xi * sin
    yi = xr * sin + xi * cos
    y = jnp.stack([yr, yi], axis=-1).reshape(x.shape)
    return y.astype(x.dtype)


def _layer_norm(x, g, b, eps=1e-5):
    m = jnp.mean(x, axis=-1, keepdims=True)
    v = jnp.var(x, axis=-1, keepdims=True)
    return (x - m) / jnp.sqrt(v + eps) * g + b


BQ = 512


def _butterfly_consts():
    import numpy as np
    j = np.arange(D)
    pmats = np.zeros((7, D, D), np.float32)
    sgns = np.zeros((7, D), np.float32)
    for st in range(7):
        h = 1 << st
        pmats[st, j ^ h, j] = 1.0
        sgns[st] = np.where(j & h, -1.0, 1.0)
    return (jnp.asarray(pmats, dtype=jnp.bfloat16),
            jnp.asarray(sgns, dtype=jnp.bfloat16))


def _rope_hadamard(x, c, s, p_ref, g_ref):
    even = (lax.broadcasted_iota(jnp.int32, x.shape, 1) % 2) == 0
    sw = jnp.where(even, pltpu.roll(x, D - 1, 1), pltpu.roll(x, 1, 1))
    y = x * c + sw * s
    yb = y.astype(jnp.bfloat16)
    for st in range(7):
        perm = lax.dot_general(yb, p_ref[st], (((1,), (0,)), ((), ())),
                               preferred_element_type=jnp.float32)
        yb = (yb * g_ref[st][None, :] + perm).astype(jnp.bfloat16)
    return yb * (D ** (-0.5))


def _scores_kernel(q_ref, c_ref, s_ref, p_ref, g_ref, kf_ref, w_ref, out_ref):
    h = pl.program_id(1)
    qf = _rope_hadamard(q_ref[...], c_ref[...], s_ref[...], p_ref, g_ref)
    s = lax.dot_general(qf, kf_ref[...],
                        (((1,), (1,)), ((), ())),
                        preferred_element_type=jnp.float32)
    s = jnp.maximum(s, 0.0) * w_ref[0, 0][:, None]

    @pl.when(h == 0)
    def _():
        out_ref[...] = s

    @pl.when(h > 0)
    def _():
        out_ref[...] += s


def _scores(qraw, cpad, spad, kf, w_h):
    p, g = _butterfly_consts()
    return pl.pallas_call(
        _scores_kernel,
        grid=(S // BQ, H),
        in_specs=[
            pl.BlockSpec((BQ, D), lambda i, h: (i, h)),
            pl.BlockSpec((BQ, D), lambda i, h: (i, 0)),
            pl.BlockSpec((BQ, D), lambda i, h: (i, 0)),
            pl.BlockSpec((7, D, D), lambda i, h: (0, 0, 0)),
            pl.BlockSpec((7, D), lambda i, h: (0, 0)),
            pl.BlockSpec((S, D), lambda i, h: (0, 0)),
            pl.BlockSpec((1, 1, BQ), lambda i, h: (h, 0, i)),
        ],
        out_specs=pl.BlockSpec((BQ, S), lambda i, h: (i, 0)),
        out_shape=jax.ShapeDtypeStruct((S, S), jnp.float32),
    )(qraw, cpad, spad, p, g, kf, w_h)


def _prep_kernel(x_ref, c_ref, s_ref, p_ref, g_ref, out_ref):
    out_ref[...] = _rope_hadamard(x_ref[...], c_ref[...], s_ref[...],
                                  p_ref, g_ref)


def _prep_k(k_ln, cpad, spad):
    p, g = _butterfly_consts()
    return pl.pallas_call(
        _prep_kernel,
        grid=(S // BQ,),
        in_specs=[
            pl.BlockSpec((BQ, D), lambda i: (i, 0)),
            pl.BlockSpec((BQ, D), lambda i: (i, 0)),
            pl.BlockSpec((BQ, D), lambda i: (i, 0)),
            pl.BlockSpec((7, D, D), lambda i: (0, 0, 0)),
            pl.BlockSpec((7, D), lambda i: (0, 0)),
        ],
        out_specs=pl.BlockSpec((BQ, D), lambda i: (i, 0)),
        out_shape=jax.ShapeDtypeStruct((S, D), jnp.bfloat16),
    )(k_ln, cpad, spad, p, g)


NW = 32
RPT = S // NW
NGRP = RPT // 16


NCH = 4
CH = S // NCH
RB = S


def _argsort_body(scores_hbm, out_hbm, rowbuf, keys, ia, h0, h1, h2, h3, sem):
    lane = lax.iota(jnp.int32, 16)
    zero16 = jnp.zeros((16,), jnp.int32)
    one16 = jnp.ones((16,), jnp.int32)
    hists = [h0, h1, h2, h3]

    def full(v):
        return jnp.full((16,), v, jnp.int32)

    wid = lax.axis_index("s") * 2 + lax.axis_index("c")

    def group(g, _):
        q0 = wid * RPT + g * 16

        copies = [
            pltpu.async_copy(scores_hbm.at[q0 + r],
                             rowbuf.at[pl.ds(r * RB, S)], sem)
            for r in range(16)
        ]
        for c in copies:
            c.wait()

        def zero_hists():
            def zbody(b, _):
                for u in range(4):
                    for hc in hists:
                        hc[pl.ds((b * 4 + u) * 16, 16)] = zero16
                return 0

            lax.fori_loop(0, 64, zbody, 0)

        def scan_hists():
            def sbody(b, run):
                hs = [hc[pl.ds(b * 16, 16)] for hc in hists]
                for c, hc in enumerate(hists):
                    hc[pl.ds(b * 16, 16)] = run
                    run = run + hs[c]
                return run

            lax.fori_loop(0, 256, sbody, zero16)

        zero_hists()

        def tbody(i, _):
            pairs = [(u, c) for u in range(2) for c in range(NCH)]
            jvs = [full(c * CH)
                   + jnp.bitwise_and(full(i * 2 + u) + lane, CH - 1)
                   for u, c in pairs]
            us = [plsc.load_gather(rowbuf, [lane * RB + jv]) for jv in jvs]
            ks = []
            for u in us:
                m = lax.shift_right_arithmetic(u, 31)
                xorv = jnp.bitwise_xor(
                    jnp.bitwise_or(m, jnp.int32(-2147483648)), jnp.int32(-1))
                ks.append(jnp.bitwise_xor(u, xorv))
            for jv, kv in zip(jvs, ks):
                plsc.store_scatter(keys, [jv * 16 + lane], kv)
            for (u, c), kv in zip(pairs, ks):
                dd = jnp.bitwise_and(kv, 255) * 16 + lane
                plsc.addupdate_scatter(hists[c], [dd], one16)
            return 0

        lax.fori_loop(0, CH // 2, tbody, 0)
        scan_hists()

        def p0body(i, _):
            for u in range(2):
                j = i * 2 + u
                kvs = [keys[pl.ds((j + c * CH) * 16, 16)] for c in range(NCH)]
                dds = [jnp.bitwise_and(kv, 255) * 16 + lane for kv in kvs]
                offs = [plsc.load_gather(hists[c], [dds[c]])
                        for c in range(NCH)]
                vals = [jnp.bitwise_or(
                            full(j + c * CH),
                            lax.shift_left(jnp.bitwise_and(kvs[c], 0xFF00), 8))
                        for c in range(NCH)]
                for c in range(NCH):
                    plsc.store_scatter(ia, [offs[c] * 16 + lane], vals[c])
                for c in range(NCH):
                    plsc.addupdate_scatter(hists[c], [dds[c]], one16)
            return 0

        lax.fori_loop(0, CH // 2, p0body, 0)

        for p, (src, dst) in enumerate(
                [(ia, rowbuf), (rowbuf, ia), (ia, None)], start=1):
            zero_hists()

            def hbody(i, _, src=src):
                pairs = [(u, c) for u in range(2) for c in range(NCH)]
                vals = [src[pl.ds((i * 2 + u + c * CH) * 16, 16)]
                        for u, c in pairs]
                dds = [lax.shift_right_logical(v, 16) * 16 + lane
                       for v in vals]
                for (u, c), dd in zip(pairs, dds):
                    plsc.addupdate_scatter(hists[c], [dd], one16)
                return 0

            lax.fori_loop(0, CH // 2, hbody, 0)
            scan_hists()

            def pbody(i, _, p=p, src=src, dst=dst):
                for u in range(2):
                    j = i * 2 + u
                    vals = [src[pl.ds((j + c * CH) * 16, 16)]
                            for c in range(NCH)]
                    dds = [lax.shift_right_logical(v, 16) * 16 + lane
                           for v in vals]
                    ixs = [jnp.bitwise_and(v, S - 1) for v in vals]
                    offs = [plsc.load_gather(hists[c], [dds[c]])
                            for c in range(NCH)]
                    if dst is None:
                        for c in range(NCH):
                            plsc.store_scatter(rowbuf, [lane * RB + offs[c]],
                                               ixs[c])
                    else:
                        kvs = [plsc.load_gather(keys, [ix * 16 + lane])
                               for ix in ixs]
                        sh = 0 if p == 1 else 8
                        nvals = [jnp.bitwise_or(
                                     ixs[c],
                                     jnp.bitwise_and(
                                         lax.shift_right_logical(kvs[c], sh),
                                         0xFF0000))
                                 for c in range(NCH)]
                        for c in range(NCH):
                            plsc.store_scatter(dst, [offs[c] * 16 + lane],
                                               nvals[c])
                    for c in range(NCH):
                        plsc.addupdate_scatter(hists[c], [dds[c]], one16)
                return 0

            lax.fori_loop(0, CH // 2, pbody, 0)

        copies = [
            pltpu.async_copy(rowbuf.at[pl.ds(r * RB, S)],
                             out_hbm.at[q0 + r], sem)
            for r in range(16)
        ]
        for c in copies:
            c.wait()
        return 0

    lax.fori_loop(0, NGRP, group, 0)


def _argsort_desc(scores_bits):
    f = pl.kernel(
        _argsort_body,
        out_type=jax.ShapeDtypeStruct((S, S), jnp.int32),
        mesh=plsc.VectorSubcoreMesh(core_axis_name="c", subcore_axis_name="s"),
        scratch_types=[
            pltpu.VMEM((16 * S,), jnp.int32),
            pltpu.VMEM((16 * S,), jnp.int32),
            pltpu.VMEM((16 * S,), jnp.int32),
            pltpu.VMEM((256 * 16,), jnp.int32),
            pltpu.VMEM((256 * 16,), jnp.int32),
            pltpu.VMEM((256 * 16,), jnp.int32),
            pltpu.VMEM((256 * 16,), jnp.int32),
            pltpu.SemaphoreType.DMA,
        ],
        compiler_params=pltpu.CompilerParams(needs_layout_passes=False),
    )
    return f(scores_bits)


def kernel(x, q_resid, freqs_cis, Wq_b, Wk, k_norm_weight, k_norm_bias, Wweights):
    softmax_scale = D ** (-0.5)
    qraw = (q_resid @ Wq_b.T).reshape(S, H * D)
    k_ln = _layer_norm(x @ Wk.T, k_norm_weight, k_norm_bias)[0]
    weights = (x @ Wweights.T).astype(jnp.float32) * (H ** (-0.5)) * softmax_scale

    cos = jnp.cos(freqs_cis)
    sin = jnp.sin(freqs_cis)
    c_rep = jnp.repeat(cos, 2, axis=1)
    s_alt = jnp.stack([-sin, sin], axis=-1).reshape(S, ROPE)
    ones = jnp.ones((S, NOPE), jnp.float32)
    zeros = jnp.zeros((S, NOPE), jnp.float32)
    cpad = jnp.concatenate([ones, c_rep], axis=1)
    spad = jnp.concatenate([zeros, s_alt], axis=1)

    kf = _prep_k(k_ln, cpad, spad)
    w_h = jnp.transpose(weights[0], (1, 0))[:, None, :]

    sc = _scores(qraw, cpad, spad, kf, w_h)
    scores_bits = lax.bitcast_convert_type(sc, jnp.int32)
    topk_indices = _argsort_desc(scores_bits)
    return topk_indices[None]


# speedup vs baseline: 5.0923x; 1.2095x over previous
import functools

import jax
import jax.numpy as jnp
from jax import lax
from jax.experimental import pallas as pl
from jax.experimental.pallas import tpu as pltpu
from jax.experimental.pallas import tpu_sc as plsc

B, S, HID = 1, 2048, 2048
H, D, ROPE, NOPE, QLORA, TOPK = 16, 128, 64, 64, 1536, 2048


def _hadamard_transform(x, scale):
    shp = x.shape
    n = shp[-1]
    y = x.reshape(-1, n)
    h = 1
    while h < n:
        y = y.reshape(-1, n // (2 * h), 2, h)
        a = y[:, :, 0, :]
        b = y[:, :, 1, :]
        y = jnp.stack([a + b, a - b], axis=2)
        y = y.reshape(-1, n)
        h *= 2
    return (y * scale).reshape(shp)


def _rotate_activation(x):
    xb = x.astype(jnp.bfloat16)
    return _hadamard_transform(xb, xb.shape[-1] ** (-0.5))


def _apply_rope(x, angles):
    cos = jnp.cos(angles)
    sin = jnp.sin(angles)
    if x.ndim == 4:
        cos = cos[None, :, None, :]
        sin = sin[None, :, None, :]
    else:
        cos = cos[None, :, :]
        sin = sin[None, :, :]
    xr = x[..., 0::2].astype(jnp.float32)
    xi = x[..., 1::2].astype(jnp.float32)
    yr = xr * cos - xi * sin
    yi = xr * sin + xi * cos
    y = jnp.stack([yr, yi], axis=-1).reshape(x.shape)
    return y.astype(x.dtype)


def _layer_norm(x, g, b, eps=1e-5):
    m = jnp.mean(x, axis=-1, keepdims=True)
    v = jnp.var(x, axis=-1, keepdims=True)
    return (x - m) / jnp.sqrt(v + eps) * g + b


BQ = 512


def _butterfly_consts():
    import numpy as np
    j = np.arange(D)
    pmats = np.zeros((7, D, D), np.float32)
    sgns = np.zeros((7, D), np.float32)
    for st in range(7):
        h = 1 << st
        pmats[st, j ^ h, j] = 1.0
        sgns[st] = np.where(j & h, -1.0, 1.0)
    return (jnp.asarray(pmats, dtype=jnp.bfloat16),
            jnp.asarray(sgns, dtype=jnp.bfloat16))


def _rope_hadamard(x, c, s, p_ref, g_ref):
    even = (lax.broadcasted_iota(jnp.int32, x.shape, 1) % 2) == 0
    sw = jnp.where(even, pltpu.roll(x, D - 1, 1), pltpu.roll(x, 1, 1))
    y = x * c + sw * s
    yb = y.astype(jnp.bfloat16)
    for st in range(7):
        perm = lax.dot_general(yb, p_ref[st], (((1,), (0,)), ((), ())),
                               preferred_element_type=jnp.float32)
        yb = (yb * g_ref[st][None, :] + perm).astype(jnp.bfloat16)
    return yb * (D ** (-0.5))


def _scores_kernel(q_ref, c_ref, s_ref, p_ref, g_ref, kf_ref, w_ref, out_ref):
    h = pl.program_id(1)
    qf = _rope_hadamard(q_ref[...], c_ref[...], s_ref[...], p_ref, g_ref)
    s = lax.dot_general(qf, kf_ref[...],
                        (((1,), (1,)), ((), ())),
                        preferred_element_type=jnp.float32)
    s = jnp.maximum(s, 0.0) * w_ref[0, 0][:, None]

    @pl.when(h == 0)
    def _():
        out_ref[...] = s

    @pl.when(h > 0)
    def _():
        out_ref[...] += s


def _scores_chunk(qraw, cpad, spad, p, g, kf, w_h, t):
    return pl.pallas_call(
        _scores_kernel,
        grid=(1, H),
        in_specs=[
            pl.BlockSpec((BQ, D), lambda i, h: (t, h)),
            pl.BlockSpec((BQ, D), lambda i, h: (t, 0)),
            pl.BlockSpec((BQ, D), lambda i, h: (t, 0)),
            pl.BlockSpec((7, D, D), lambda i, h: (0, 0, 0)),
            pl.BlockSpec((7, D), lambda i, h: (0, 0)),
            pl.BlockSpec((S, D), lambda i, h: (0, 0)),
            pl.BlockSpec((1, 1, BQ), lambda i, h: (h, 0, t)),
        ],
        out_specs=pl.BlockSpec((BQ, S), lambda i, h: (i, 0)),
        out_shape=jax.ShapeDtypeStruct((BQ, S), jnp.float32),
    )(qraw, cpad, spad, p, g, kf, w_h)


def _prep_kernel(x_ref, c_ref, s_ref, p_ref, g_ref, out_ref):
    out_ref[...] = _rope_hadamard(x_ref[...], c_ref[...], s_ref[...],
                                  p_ref, g_ref)


def _prep_k(k_ln, cpad, spad):
    p, g = _butterfly_consts()
    return pl.pallas_call(
        _prep_kernel,
        grid=(S // BQ,),
        in_specs=[
            pl.BlockSpec((BQ, D), lambda i: (i, 0)),
            pl.BlockSpec((BQ, D), lambda i: (i, 0)),
            pl.BlockSpec((BQ, D), lambda i: (i, 0)),
            pl.BlockSpec((7, D, D), lambda i: (0, 0, 0)),
            pl.BlockSpec((7, D), lambda i: (0, 0)),
        ],
        out_specs=pl.BlockSpec((BQ, D), lambda i: (i, 0)),
        out_shape=jax.ShapeDtypeStruct((S, D), jnp.bfloat16),
    )(k_ln, cpad, spad, p, g)


NW = 32
RPT = S // NW
NGRP = RPT // 16


NCH = 4
CH = S // NCH
RB = S


def _argsort_body(ngrp, scores_hbm, out_hbm, rowbuf, keys, ia, h0, h1, h2, h3,
                  sem):
    lane = lax.iota(jnp.int32, 16)
    zero16 = jnp.zeros((16,), jnp.int32)
    one16 = jnp.ones((16,), jnp.int32)
    hists = [h0, h1, h2, h3]

    def full(v):
        return jnp.full((16,), v, jnp.int32)

    wid = lax.axis_index("s") * 2 + lax.axis_index("c")

    def group(g, _):
        q0 = wid * (16 * ngrp) + g * 16

        copies = [
            pltpu.async_copy(scores_hbm.at[q0 + r],
                             rowbuf.at[pl.ds(r * RB, S)], sem)
            for r in range(16)
        ]
        for c in copies:
            c.wait()

        def zero_hists():
            def zbody(b, _):
                for u in range(4):
                    for hc in hists:
                        hc[pl.ds((b * 4 + u) * 16, 16)] = zero16
                return 0

            lax.fori_loop(0, 64, zbody, 0)

        def scan_hists():
            def sbody(b, run):
                hs = [hc[pl.ds(b * 16, 16)] for hc in hists]
                for c, hc in enumerate(hists):
                    hc[pl.ds(b * 16, 16)] = run
                    run = run + hs[c]
                return run

            lax.fori_loop(0, 256, sbody, zero16)

        zero_hists()

        def tbody(i, _):
            pairs = [(u, c) for u in range(2) for c in range(NCH)]
            jvs = [full(c * CH)
                   + jnp.bitwise_and(full(i * 2 + u) + lane, CH - 1)
                   for u, c in pairs]
            us = [plsc.load_gather(rowbuf, [lane * RB + jv]) for jv in jvs]
            ks = []
            for u in us:
                m = lax.shift_right_arithmetic(u, 31)
                xorv = jnp.bitwise_xor(
                    jnp.bitwise_or(m, jnp.int32(-2147483648)), jnp.int32(-1))
                ks.append(jnp.bitwise_xor(u, xorv))
            for jv, kv in zip(jvs, ks):
                plsc.store_scatter(keys, [jv * 16 + lane], kv)
            for (u, c), kv in zip(pairs, ks):
                dd = jnp.bitwise_and(kv, 255) * 16 + lane
                plsc.addupdate_scatter(hists[c], [dd], one16)
            return 0

        lax.fori_loop(0, CH // 2, tbody, 0)
        scan_hists()

        def p0body(i, _):
            for u in range(2):
                j = i * 2 + u
                kvs = [keys[pl.ds((j + c * CH) * 16, 16)] for c in range(NCH)]
                dds = [jnp.bitwise_and(kv, 255) * 16 + lane for kv in kvs]
                offs = [plsc.load_gather(hists[c], [dds[c]])
                        for c in range(NCH)]
                vals = [jnp.bitwise_or(
                            full(j + c * CH),
                            lax.shift_left(jnp.bitwise_and(kvs[c], 0xFF00), 8))
                        for c in range(NCH)]
                for c in range(NCH):
                    plsc.store_scatter(ia, [offs[c] * 16 + lane], vals[c])
                for c in range(NCH):
                    plsc.addupdate_scatter(hists[c], [dds[c]], one16)
            return 0

        lax.fori_loop(0, CH // 2, p0body, 0)

        for p, (src, dst) in enumerate(
                [(ia, rowbuf), (rowbuf, ia), (ia, None)], start=1):
            zero_hists()

            def hbody(i, _, src=src):
                pairs = [(u, c) for u in range(2) for c in range(NCH)]
                vals = [src[pl.ds((i * 2 + u + c * CH) * 16, 16)]
                        for u, c in pairs]
                dds = [lax.shift_right_logical(v, 16) * 16 + lane
                       for v in vals]
                for (u, c), dd in zip(pairs, dds):
                    plsc.addupdate_scatter(hists[c], [dd], one16)
                return 0

            lax.fori_loop(0, CH // 2, hbody, 0)
            scan_hists()

            def pbody(i, _, p=p, src=src, dst=dst):
                for u in range(2):
                    j = i * 2 + u
                    vals = [src[pl.ds((j + c * CH) * 16, 16)]
                            for c in range(NCH)]
                    dds = [lax.shift_right_logical(v, 16) * 16 + lane
                           for v in vals]
                    ixs = [jnp.bitwise_and(v, S - 1) for v in vals]
                    offs = [plsc.load_gather(hists[c], [dds[c]])
                            for c in range(NCH)]
                    if dst is None:
                        for c in range(NCH):
                            plsc.store_scatter(rowbuf, [lane * RB + offs[c]],
                                               ixs[c])
                    else:
                        kvs = [plsc.load_gather(keys, [ix * 16 + lane])
                               for ix in ixs]
                        sh = 0 if p == 1 else 8
                        nvals = [jnp.bitwise_or(
                                     ixs[c],
                                     jnp.bitwise_and(
                                         lax.shift_right_logical(kvs[c], sh),
                                         0xFF0000))
                                 for c in range(NCH)]
                        for c in range(NCH):
                            plsc.store_scatter(dst, [offs[c] * 16 + lane],
                                               nvals[c])
                    for c in range(NCH):
                        plsc.addupdate_scatter(hists[c], [dds[c]], one16)
                return 0

            lax.fori_loop(0, CH // 2, pbody, 0)

        copies = [
            pltpu.async_copy(rowbuf.at[pl.ds(r * RB, S)],
                             out_hbm.at[q0 + r], sem)
            for r in range(16)
        ]
        for c in copies:
            c.wait()
        return 0

    lax.fori_loop(0, ngrp, group, 0)


def _argsort_desc(scores_bits):
    nr = scores_bits.shape[0]
    ngrp = nr // (NW * 16)
    f = pl.kernel(
        functools.partial(_argsort_body, ngrp),
        out_type=jax.ShapeDtypeStruct((nr, S), jnp.int32),
        mesh=plsc.VectorSubcoreMesh(core_axis_name="c", subcore_axis_name="s"),
        scratch_types=[
            pltpu.VMEM((16 * S,), jnp.int32),
            pltpu.VMEM((16 * S,), jnp.int32),
            pltpu.VMEM((16 * S,), jnp.int32),
            pltpu.VMEM((256 * 16,), jnp.int32),
            pltpu.VMEM((256 * 16,), jnp.int32),
            pltpu.VMEM((256 * 16,), jnp.int32),
            pltpu.VMEM((256 * 16,), jnp.int32),
            pltpu.SemaphoreType.DMA,
        ],
        compiler_params=pltpu.CompilerParams(needs_layout_passes=False),
    )
    return f(scores_bits)


def kernel(x, q_resid, freqs_cis, Wq_b, Wk, k_norm_weight, k_norm_bias, Wweights):
    softmax_scale = D ** (-0.5)
    qraw = (q_resid @ Wq_b.T).reshape(S, H * D)
    k_ln = _layer_norm(x @ Wk.T, k_norm_weight, k_norm_bias)[0]
    weights = (x @ Wweights.T).astype(jnp.float32) * (H ** (-0.5)) * softmax_scale

    cos = jnp.cos(freqs_cis)
    sin = jnp.sin(freqs_cis)
    c_rep = jnp.repeat(cos, 2, axis=1)
    s_alt = jnp.stack([-sin, sin], axis=-1).reshape(S, ROPE)
    ones = jnp.ones((S, NOPE), jnp.float32)
    zeros = jnp.zeros((S, NOPE), jnp.float32)
    cpad = jnp.concatenate([ones, c_rep], axis=1)
    spad = jnp.concatenate([zeros, s_alt], axis=1)

    kf = _prep_k(k_ln, cpad, spad)
    w_h = jnp.transpose(weights[0], (1, 0))[:, None, :]

    p, g = _butterfly_consts()
    outs = []
    for t in range(S // BQ):
        sc = _scores_chunk(qraw, cpad, spad, p, g, kf, w_h, t)
        bits = lax.bitcast_convert_type(sc, jnp.int32)
        outs.append(_argsort_desc(bits))
    topk_indices = jnp.concatenate(outs, axis=0)
    return topk_indices[None]
